# Initial kernel scaffold; baseline (speedup 1.0000x reference)
import jax
import jax.numpy as jnp
from jax.experimental import pallas as pl


def kernel(node_feature, neighbor_features, W1, b1):
    raise NotImplementedError("write your pallas kernel here")


# speedup vs baseline: 1.0090x; 1.0090x over previous
import jax
import jax.numpy as jnp
from jax.experimental import pallas as pl


def _agg_kernel(node_ref, nbr_ref, w1_ref, b1_ref, out_ref):
    nbr = nbr_ref[...]
    mean = jnp.mean(nbr, axis=0, keepdims=True)
    combined = jnp.concatenate([node_ref[...], mean], axis=1)
    out = jax.lax.dot_general(
        combined, w1_ref[...],
        dimension_numbers=(((1,), (1,)), ((), ())),
        preferred_element_type=jnp.float32,
    )
    out_ref[...] = jnp.maximum(out + b1_ref[...], 0.0)


@jax.jit
def kernel(node_feature, neighbor_features, W1, b1):
    in_dim = node_feature.shape[0]
    out_dim = b1.shape[0]
    out = pl.pallas_call(
        _agg_kernel,
        out_shape=jax.ShapeDtypeStruct((1, out_dim), jnp.float32),
    )(
        node_feature.reshape(1, in_dim),
        neighbor_features,
        W1,
        b1.reshape(1, out_dim),
    )
    return out.reshape(out_dim)
